# baseline (speedup 1.0000x reference)
import jax
import jax.numpy as jnp
from jax import lax
from jax.experimental import pallas as pl
from jax.experimental.pallas import tpu as pltpu
from jax.experimental.pallas import tpu_sc as plsc

_F = 128
_Q = 65536
_B = 4096

_NC = 2
_NS = 16
_NW = _NC * _NS
_L = 16

_CPW = _B // _NW
_NB = 16
_BH = _F // _NB


def _slab_body(feat_hbm, slab_hbm, fbuf, tbuf, fsem, osem):
    cid = lax.axis_index("c")
    sid = lax.axis_index("s")
    wid = sid * _NC + cid
    c0 = wid * _CPW

    pltpu.async_copy(feat_hbm.at[pl.ds(c0, _CPW), :], fbuf, fsem).wait()

    iot = jnp.arange(_L, dtype=jnp.int32)

    def _tbody(m, carry):
        j0 = (m // 8) * _L
        f0 = (m % 8) * _L
        jv = j0 + iot
        for d in range(_L):
            fv = f0 + ((iot + d) & (_L - 1))
            v = plsc.load_gather(fbuf, [jv, fv])
            plsc.store_scatter(tbuf, [fv // _BH, fv % _BH, jv], v)
        return carry

    lax.fori_loop(0, (_CPW // _L) * (_F // _L), _tbody, 0)

    ods = [
        pltpu.async_copy(
            tbuf.at[b],
            slab_hbm.at[pl.ds(b * _BH, _BH), pl.ds(c0, _CPW)],
            osem,
        )
        for b in range(_NB)
    ]
    for d in ods:
        d.wait()


_slab_kernel = pl.kernel(
    _slab_body,
    out_type=jax.ShapeDtypeStruct((_F, _B), jnp.float32),
    mesh=plsc.VectorSubcoreMesh(core_axis_name="c", subcore_axis_name="s"),
    compiler_params=pltpu.CompilerParams(needs_layout_passes=False),
    scratch_types=[
        pltpu.VMEM((_CPW, _F), jnp.float32),
        pltpu.VMEM((_NB, _BH, _CPW), jnp.float32),
        pltpu.SemaphoreType.DMA,
        pltpu.SemaphoreType.DMA,
    ],
)


_CCH = 1536
_NCH = (_Q - _B) // _CCH
_TNBUF = 16
_TAHEAD = 6
_MCH = 1024
_MNCH = _B // _MCH
_MNBUF = 4


def _copy_body(q_ref, o_ref, *rest):
    bufs = rest[:_TNBUF]
    rsem, wsem = rest[_TNBUF:]

    def _src(k):
        return q_ref.at[:, pl.ds(_B + k * _CCH, _CCH)]

    def _dst(k):
        return o_ref.at[:, pl.ds(_B + k * _CCH, _CCH)]

    rd = {}
    wd = {}
    for i in range(_TAHEAD + 1):
        rd[i] = pltpu.make_async_copy(_src(i), bufs[i], rsem)
        rd[i].start()
    for k in range(_NCH):
        s = k % _TNBUF
        kn = k + _TAHEAD + 1
        if kn < _NCH:
            sn = kn % _TNBUF
            if kn >= _TNBUF:
                wd[sn].wait()
            rd[sn] = pltpu.make_async_copy(_src(kn), bufs[sn], rsem)
            rd[sn].start()
        rd[s].wait()
        wd[s] = pltpu.make_async_copy(bufs[s], _dst(k), wsem)
        wd[s].start()
    for k in range(max(0, _NCH - _TNBUF), _NCH):
        wd[k % _TNBUF].wait()


def _merge_body(slab_ref, prev_ref, o_ref, *rest):
    del prev_ref
    bufs = rest[:_MNBUF]
    rsem, wsem = rest[_MNBUF:]
    rd = {}
    wd = {}
    for i in range(_MNCH):
        rd[i] = pltpu.make_async_copy(
            slab_ref.at[:, pl.ds(i * _MCH, _MCH)], bufs[i], rsem
        )
        rd[i].start()
    for k in range(_MNCH):
        rd[k].wait()
        wd[k] = pltpu.make_async_copy(
            bufs[k], o_ref.at[:, pl.ds(k * _MCH, _MCH)], wsem
        )
        wd[k].start()
    for k in range(_MNCH):
        wd[k].wait()


def kernel(features, queue):
    slab = _slab_kernel(features)
    out1 = pl.pallas_call(
        _copy_body,
        in_specs=[pl.BlockSpec(memory_space=pl.ANY)],
        out_specs=pl.BlockSpec(memory_space=pl.ANY),
        out_shape=jax.ShapeDtypeStruct((_F, _Q), jnp.float32),
        scratch_shapes=(
            [pltpu.VMEM((_F, _CCH), jnp.float32) for _ in range(_TNBUF)]
            + [pltpu.SemaphoreType.DMA, pltpu.SemaphoreType.DMA]
        ),
    )(queue)
    out = pl.pallas_call(
        _merge_body,
        in_specs=[
            pl.BlockSpec(memory_space=pl.ANY),
            pl.BlockSpec(memory_space=pl.ANY),
        ],
        out_specs=pl.BlockSpec(memory_space=pl.ANY),
        out_shape=jax.ShapeDtypeStruct((_F, _Q), jnp.float32),
        scratch_shapes=(
            [pltpu.VMEM((_F, _MCH), jnp.float32) for _ in range(_MNBUF)]
            + [pltpu.SemaphoreType.DMA, pltpu.SemaphoreType.DMA]
        ),
        input_output_aliases={1: 0},
    )(slab, out1)
    return out


# speedup vs baseline: 1.0054x; 1.0054x over previous
import jax
import jax.numpy as jnp
from jax import lax
from jax.experimental import pallas as pl
from jax.experimental.pallas import tpu as pltpu
from jax.experimental.pallas import tpu_sc as plsc

_F = 128
_Q = 65536
_B = 4096

_NC = 2
_NS = 16
_NW = _NC * _NS
_L = 16

_CPW = _B // _NW
_NB = 16
_BH = _F // _NB


def _slab_body(feat_hbm, slab_hbm, fbuf, tbuf, fsem, osem):
    cid = lax.axis_index("c")
    sid = lax.axis_index("s")
    wid = sid * _NC + cid
    c0 = wid * _CPW

    pltpu.async_copy(feat_hbm.at[pl.ds(c0, _CPW), :], fbuf, fsem).wait()

    iot = jnp.arange(_L, dtype=jnp.int32)

    def _tbody(m, carry):
        j0 = (m // 8) * _L
        f0 = (m % 8) * _L
        jv = j0 + iot
        for d in range(_L):
            fv = f0 + ((iot + d) & (_L - 1))
            v = plsc.load_gather(fbuf, [jv, fv])
            plsc.store_scatter(tbuf, [fv // _BH, fv % _BH, jv], v)
        return carry

    lax.fori_loop(0, (_CPW // _L) * (_F // _L), _tbody, 0)

    ods = [
        pltpu.async_copy(
            tbuf.at[b],
            slab_hbm.at[pl.ds(b * _BH, _BH), pl.ds(c0, _CPW)],
            osem,
        )
        for b in range(_NB)
    ]
    for d in ods:
        d.wait()


_slab_kernel = pl.kernel(
    _slab_body,
    out_type=jax.ShapeDtypeStruct((_F, _B), jnp.float32),
    mesh=plsc.VectorSubcoreMesh(core_axis_name="c", subcore_axis_name="s"),
    compiler_params=pltpu.CompilerParams(needs_layout_passes=False),
    scratch_types=[
        pltpu.VMEM((_CPW, _F), jnp.float32),
        pltpu.VMEM((_NB, _BH, _CPW), jnp.float32),
        pltpu.SemaphoreType.DMA,
        pltpu.SemaphoreType.DMA,
    ],
)


_CCH = 2048
_NCH = (_Q - _B) // _CCH
_TNBUF = 12
_TAHEAD = 5
_MCH = 1024
_MNCH = _B // _MCH
_MNBUF = 4


def _copy_body(q_ref, o_ref, *rest):
    bufs = rest[:_TNBUF]
    rsem, wsem = rest[_TNBUF:]

    def _src(k):
        return q_ref.at[:, pl.ds(_B + k * _CCH, _CCH)]

    def _dst(k):
        return o_ref.at[:, pl.ds(_B + k * _CCH, _CCH)]

    rd = {}
    wd = {}
    for i in range(_TAHEAD + 1):
        rd[i] = pltpu.make_async_copy(_src(i), bufs[i], rsem)
        rd[i].start()
    for k in range(_NCH):
        s = k % _TNBUF
        kn = k + _TAHEAD + 1
        if kn < _NCH:
            sn = kn % _TNBUF
            if kn >= _TNBUF:
                wd[sn].wait()
            rd[sn] = pltpu.make_async_copy(_src(kn), bufs[sn], rsem)
            rd[sn].start()
        rd[s].wait()
        wd[s] = pltpu.make_async_copy(bufs[s], _dst(k), wsem)
        wd[s].start()
    for k in range(max(0, _NCH - _TNBUF), _NCH):
        wd[k % _TNBUF].wait()


def _merge_body(slab_ref, prev_ref, o_ref, *rest):
    del prev_ref
    bufs = rest[:_MNBUF]
    rsem, wsem = rest[_MNBUF:]
    rd = {}
    wd = {}
    for i in range(_MNCH):
        rd[i] = pltpu.make_async_copy(
            slab_ref.at[:, pl.ds(i * _MCH, _MCH)], bufs[i], rsem
        )
        rd[i].start()
    for k in range(_MNCH):
        rd[k].wait()
        wd[k] = pltpu.make_async_copy(
            bufs[k], o_ref.at[:, pl.ds(k * _MCH, _MCH)], wsem
        )
        wd[k].start()
    for k in range(_MNCH):
        wd[k].wait()


def kernel(features, queue):
    slab = _slab_kernel(features)
    out1 = pl.pallas_call(
        _copy_body,
        in_specs=[pl.BlockSpec(memory_space=pl.ANY)],
        out_specs=pl.BlockSpec(memory_space=pl.ANY),
        out_shape=jax.ShapeDtypeStruct((_F, _Q), jnp.float32),
        scratch_shapes=(
            [pltpu.VMEM((_F, _CCH), jnp.float32) for _ in range(_TNBUF)]
            + [pltpu.SemaphoreType.DMA, pltpu.SemaphoreType.DMA]
        ),
    )(queue)
    out = pl.pallas_call(
        _merge_body,
        in_specs=[
            pl.BlockSpec(memory_space=pl.ANY),
            pl.BlockSpec(memory_space=pl.ANY),
        ],
        out_specs=pl.BlockSpec(memory_space=pl.ANY),
        out_shape=jax.ShapeDtypeStruct((_F, _Q), jnp.float32),
        scratch_shapes=(
            [pltpu.VMEM((_F, _MCH), jnp.float32) for _ in range(_MNBUF)]
            + [pltpu.SemaphoreType.DMA, pltpu.SemaphoreType.DMA]
        ),
        input_output_aliases={1: 0},
    )(slab, out1)
    return out
